# baseline (speedup 1.0000x reference)
import functools

import jax
import jax.numpy as jnp
from jax.experimental import pallas as pl
from jax.experimental.pallas import tpu as pltpu

_CHUNK = 256


def _dma_body(batch, seq, d, chunk, w_hbm, o_hbm, vmem, in_sems, out_sems):
    n = seq // chunk

    def read(i):
        return pltpu.make_async_copy(
            w_hbm.at[pl.ds(i * chunk, chunk)],
            vmem.at[pl.ds(i * chunk, chunk)], in_sems.at[i])

    def write(i, b):
        return pltpu.make_async_copy(
            vmem.at[pl.ds(i * chunk, chunk)],
            o_hbm.at[b, pl.ds(i * chunk, chunk)], out_sems.at[i])

    for i in range(n):
        read(i).start()
    for i in range(n):
        read(i).wait()
        for b in range(batch):
            write(i, b).start()
    for i in range(n):
        for b in range(batch):
            write(i, b).wait()


def kernel(tokens, W_pos):
    batch, seq = tokens.shape
    d = W_pos.shape[-1]
    pos = W_pos[:seq]
    chunk = min(_CHUNK, seq)
    n = seq // chunk
    return pl.pallas_call(
        functools.partial(_dma_body, batch, seq, d, chunk),
        in_specs=[pl.BlockSpec(memory_space=pl.ANY)],
        out_specs=pl.BlockSpec(memory_space=pl.ANY),
        out_shape=jax.ShapeDtypeStruct((batch, seq, d), W_pos.dtype),
        scratch_shapes=[
            pltpu.VMEM((seq, d), W_pos.dtype),
            pltpu.SemaphoreType.DMA((n,)),
            pltpu.SemaphoreType.DMA((n,)),
        ],
    )(pos)
